# baseline (speedup 1.0000x reference)
import functools

import jax
import jax.numpy as jnp
from jax import lax
from jax.experimental import pallas as pl
from jax.experimental.pallas import tpu as pltpu
from jax.experimental.pallas import tpu_sc as plsc

_VOCAB = 1000000
_CTX = 200
_EMB = 64
_BATCH = 1024

_NC = 2
_NS = 16
_NW = _NC * _NS
_ROWS = _BATCH * _CTX
_RPW = _ROWS // _NW
_CHUNK = 4 * _CTX
_NCHUNK = _RPW // _CHUNK
_GL = 100
_NG = _CHUNK // _GL
_LANES = 16


def _pos_encoding():
    positions = jnp.arange(_CTX, dtype=jnp.float32)
    indices = jnp.arange(_EMB // 2, dtype=jnp.float32)
    scaling_factor = 10000 ** (2 * indices / _EMB)
    angles = positions[:, None] / scaling_factor
    pe = jnp.zeros((_CTX, _EMB), dtype=jnp.float32)
    pe = pe.at[:, 0::2].set(jnp.sin(angles))
    pe = pe.at[:, 1::2].set(jnp.cos(angles))
    return pe


def _body(x_hbm, pe_hbm, table_hbm, out_hbm, idx_v, rows_v, pe_v, gsem):
    wid = lax.axis_index("s") * _NC + lax.axis_index("c")
    pltpu.sync_copy(pe_hbm, pe_v)

    def chunk_body(c, carry):
        xrow = wid * (_RPW // _GL) + c * _NG
        pltpu.sync_copy(x_hbm.at[pl.ds(xrow, _NG), :], idx_v)
        copies = [
            pltpu.async_copy(
                table_hbm.at[idx_v.at[j]],
                rows_v.at[pl.ds(j * _GL, _GL), :],
                gsem,
            )
            for j in range(_NG)
        ]
        for cp in copies:
            cp.wait()

        def pos_body(p, pcarry):
            for g in range(_EMB // _LANES):
                sl = pl.ds(g * _LANES, _LANES)
                pv = pe_v[p, sl]
                for r in range(_CHUNK // _CTX):
                    i = r * _CTX + p
                    rows_v[i, sl] = rows_v[i, sl] + pv
            return pcarry

        lax.fori_loop(0, _CTX, pos_body, 0)
        base = wid * _RPW + c * _CHUNK
        pltpu.sync_copy(rows_v, out_hbm.at[pl.ds(base, _CHUNK), :])
        return carry

    lax.fori_loop(0, _NCHUNK, chunk_body, 0)


_mesh = plsc.VectorSubcoreMesh(core_axis_name="c", subcore_axis_name="s")

_emb_call = functools.partial(
    pl.kernel,
    mesh=_mesh,
    out_type=jax.ShapeDtypeStruct((_ROWS, _EMB), jnp.float32),
    compiler_params=pltpu.CompilerParams(use_tc_tiling_on_sc=False),
    scratch_types=[
        pltpu.VMEM((_NG, _GL), jnp.int32),
        pltpu.VMEM((_CHUNK, _EMB), jnp.float32),
        pltpu.VMEM((_CTX, _EMB), jnp.float32),
        pltpu.SemaphoreType.DMA,
    ],
)(_body)


@jax.jit
def kernel(x, table):
    pe = _pos_encoding()
    x2 = x.reshape(_ROWS // _GL, _GL)
    out = _emb_call(x2, pe, table)
    return out.reshape(_BATCH, _CTX, _EMB)


# speedup vs baseline: 1.0291x; 1.0291x over previous
import functools

import jax
import jax.numpy as jnp
from jax import lax
from jax.experimental import pallas as pl
from jax.experimental.pallas import tpu as pltpu
from jax.experimental.pallas import tpu_sc as plsc

_VOCAB = 1000000
_CTX = 200
_EMB = 64
_BATCH = 1024

_NC = 2
_NS = 16
_NW = _NC * _NS
_ROWS = _BATCH * _CTX
_RPW = _ROWS // _NW
_CHUNK = 4 * _CTX
_NCHUNK = _RPW // _CHUNK
_GL = 100
_NG = _CHUNK // _GL
_LANES = 16


def _pos_encoding():
    positions = jnp.arange(_CTX, dtype=jnp.float32)
    indices = jnp.arange(_EMB // 2, dtype=jnp.float32)
    scaling_factor = 10000 ** (2 * indices / _EMB)
    angles = positions[:, None] / scaling_factor
    pe = jnp.zeros((_CTX, _EMB), dtype=jnp.float32)
    pe = pe.at[:, 0::2].set(jnp.sin(angles))
    pe = pe.at[:, 1::2].set(jnp.cos(angles))
    return pe


def _body(x_hbm, pe_hbm, table_hbm, out_hbm,
          idx0, idx1, idx2, rows0, rows1, pe_v,
          isem0, isem1, isem2, gsem0, gsem1, ssem):
    wid = lax.axis_index("s") * _NC + lax.axis_index("c")
    pltpu.sync_copy(pe_hbm, pe_v)
    idx = [idx0, idx1, idx2]
    isem = [isem0, isem1, isem2]
    rows = [rows0, rows1]
    gsem = [gsem0, gsem1]

    def idx_copy(c):
        xrow = wid * (_RPW // _GL) + c * _NG
        return pltpu.async_copy(
            x_hbm.at[pl.ds(xrow, _NG), :], idx[c % 3], isem[c % 3])

    def gather_copies(c):
        buf = rows[c % 2]
        return [
            pltpu.async_copy(
                table_hbm.at[idx[c % 3].at[j]],
                buf.at[pl.ds(j * _GL, _GL), :],
                gsem[c % 2],
            )
            for j in range(_NG)
        ]

    def scatter_copy(c):
        base = wid * _RPW + c * _CHUNK
        return pltpu.async_copy(
            rows[c % 2], out_hbm.at[pl.ds(base, _CHUNK), :], ssem)

    def pe_add(c):
        buf = rows[c % 2]

        def pos_body(p, pcarry):
            for g in range(_EMB // _LANES):
                sl = pl.ds(g * _LANES, _LANES)
                pv = pe_v[p, sl]
                for r in range(_CHUNK // _CTX):
                    i = r * _CTX + p
                    buf[i, sl] = buf[i, sl] + pv
            return pcarry

        lax.fori_loop(0, _CTX, pos_body, 0)

    idx_copy(0).wait()
    gathers = {0: gather_copies(0)}
    idx_pending = {1: idx_copy(1)}
    scatters = {}

    for c in range(_NCHUNK):
        if c + 1 < _NCHUNK:
            idx_pending.pop(c + 1).wait()
            if c >= 1:
                scatters.pop(c - 1).wait()
            gathers[c + 1] = gather_copies(c + 1)
        if c + 2 < _NCHUNK:
            idx_pending[c + 2] = idx_copy(c + 2)
        for cp in gathers.pop(c):
            cp.wait()
        pe_add(c)
        scatters[c] = scatter_copy(c)

    scatters.pop(_NCHUNK - 2).wait()
    scatters.pop(_NCHUNK - 1).wait()


_mesh = plsc.VectorSubcoreMesh(core_axis_name="c", subcore_axis_name="s")

_emb_call = functools.partial(
    pl.kernel,
    mesh=_mesh,
    out_type=jax.ShapeDtypeStruct((_ROWS, _EMB), jnp.float32),
    compiler_params=pltpu.CompilerParams(use_tc_tiling_on_sc=False),
    scratch_types=[
        pltpu.VMEM((_NG, _GL), jnp.int32),
        pltpu.VMEM((_NG, _GL), jnp.int32),
        pltpu.VMEM((_NG, _GL), jnp.int32),
        pltpu.VMEM((_CHUNK, _EMB), jnp.float32),
        pltpu.VMEM((_CHUNK, _EMB), jnp.float32),
        pltpu.VMEM((_CTX, _EMB), jnp.float32),
        pltpu.SemaphoreType.DMA,
        pltpu.SemaphoreType.DMA,
        pltpu.SemaphoreType.DMA,
        pltpu.SemaphoreType.DMA,
        pltpu.SemaphoreType.DMA,
        pltpu.SemaphoreType.DMA,
    ],
)(_body)


@jax.jit
def kernel(x, table):
    pe = _pos_encoding()
    x2 = x.reshape(_ROWS // _GL, _GL)
    out = _emb_call(x2, pe, table)
    return out.reshape(_BATCH, _CTX, _EMB)


# speedup vs baseline: 1.0293x; 1.0003x over previous
import functools

import jax
import jax.numpy as jnp
from jax import lax
from jax.experimental import pallas as pl
from jax.experimental.pallas import tpu as pltpu
from jax.experimental.pallas import tpu_sc as plsc

_VOCAB = 1000000
_CTX = 200
_EMB = 64
_BATCH = 1024

_NC = 2
_NS = 16
_NW = _NC * _NS
_ROWS = _BATCH * _CTX
_BPW = _BATCH // _NW
_CB = 4
_CHUNK = _CB * _CTX
_NCHUNK = _BPW // _CB
_GL = 100
_NG = _CHUNK // _GL
_LANES = 16


def _pos_encoding():
    positions = jnp.arange(_CTX, dtype=jnp.float32)
    indices = jnp.arange(_EMB // 2, dtype=jnp.float32)
    scaling_factor = 10000 ** (2 * indices / _EMB)
    angles = positions[:, None] / scaling_factor
    pe = jnp.zeros((_CTX, _EMB), dtype=jnp.float32)
    pe = pe.at[:, 0::2].set(jnp.sin(angles))
    pe = pe.at[:, 1::2].set(jnp.cos(angles))
    return pe


def _body(x_hbm, pe_hbm, table_hbm, out_hbm,
          idx0, idx1, idx2, rows0, rows1, pe_v,
          isem0, isem1, isem2, gsem0, gsem1, ssem0, ssem1):
    wid = lax.axis_index("s") * _NC + lax.axis_index("c")
    pltpu.sync_copy(pe_hbm, pe_v)
    idx = [idx0, idx1, idx2]
    isem = [isem0, isem1, isem2]
    rows = [rows0, rows1]
    gsem = [gsem0, gsem1]
    ssem = [ssem0, ssem1]

    def idx_copy(c):
        xrow = wid * (_BPW * _CTX // _GL) + c * _NG
        return pltpu.async_copy(
            x_hbm.at[pl.ds(xrow, _NG), :], idx[c % 3], isem[c % 3])

    def gather_copies(c):
        buf = rows[c % 2]
        return [
            pltpu.async_copy(
                table_hbm.at[idx[c % 3].at[j]],
                buf.at[j // 2, pl.ds((j % 2) * _GL, _GL), :],
                gsem[c % 2],
            )
            for j in range(_NG)
        ]

    def scatter_copy(c):
        b0 = wid * _BPW + c * _CB
        return pltpu.async_copy(
            rows[c % 2], out_hbm.at[pl.ds(b0, _CB), :, :], ssem[c % 2])

    def pe_add(c):
        buf = rows[c % 2]

        def pos_body(p, pcarry):
            for g in range(_EMB // _LANES):
                sl = pl.ds(g * _LANES, _LANES)
                pv = pe_v[p, sl]
                for r in range(_CB):
                    buf[r, p, sl] = buf[r, p, sl] + pv
            return pcarry

        lax.fori_loop(0, _CTX, pos_body, 0)

    idx_copy(0).wait()
    gathers = {0: gather_copies(0)}
    idx_pending = {1: idx_copy(1)}
    scatters = {}

    for c in range(_NCHUNK):
        if c + 1 < _NCHUNK:
            idx_pending.pop(c + 1).wait()
            if c >= 1:
                scatters.pop(c - 1).wait()
            gathers[c + 1] = gather_copies(c + 1)
        if c + 2 < _NCHUNK:
            idx_pending[c + 2] = idx_copy(c + 2)
        for cp in gathers.pop(c):
            cp.wait()
        pe_add(c)
        scatters[c] = scatter_copy(c)

    scatters.pop(_NCHUNK - 2).wait()
    scatters.pop(_NCHUNK - 1).wait()


_mesh = plsc.VectorSubcoreMesh(core_axis_name="c", subcore_axis_name="s")

_emb_call = functools.partial(
    pl.kernel,
    mesh=_mesh,
    out_type=jax.ShapeDtypeStruct((_BATCH, _CTX, _EMB), jnp.float32),
    compiler_params=pltpu.CompilerParams(use_tc_tiling_on_sc=False),
    scratch_types=[
        pltpu.VMEM((_NG, _GL), jnp.int32),
        pltpu.VMEM((_NG, _GL), jnp.int32),
        pltpu.VMEM((_NG, _GL), jnp.int32),
        pltpu.VMEM((_CB, _CTX, _EMB), jnp.float32),
        pltpu.VMEM((_CB, _CTX, _EMB), jnp.float32),
        pltpu.VMEM((_CTX, _EMB), jnp.float32),
        pltpu.SemaphoreType.DMA,
        pltpu.SemaphoreType.DMA,
        pltpu.SemaphoreType.DMA,
        pltpu.SemaphoreType.DMA,
        pltpu.SemaphoreType.DMA,
        pltpu.SemaphoreType.DMA,
        pltpu.SemaphoreType.DMA,
    ],
)(_body)


@jax.jit
def kernel(x, table):
    pe = _pos_encoding()
    x2 = x.reshape(_ROWS // _GL, _GL)
    return _emb_call(x2, pe, table)
